# baseline (speedup 1.0000x reference)
import functools

import jax
import jax.numpy as jnp
from jax import lax
from jax.experimental import pallas as pl
from jax.experimental.pallas import tpu as pltpu
from jax.experimental.pallas import tpu_sc as plsc

N = 10000
E = 640000
B = 512
H = 128
PROT = 128

NSC = 2
NSUB = 16
NW = NSC * NSUB
K = 128
NCHUNK = E // K
QR = 80
NQ = N // QR
SEGS_PER_W = B // NW
PCH = 256

_mesh = plsc.VectorSubcoreMesh(core_axis_name="c", subcore_axis_name="s")

_Z16 = functools.partial(jnp.zeros, (16,), jnp.float32)


def _zero_rows(ref, nrows, ncol16):
    def body(r, _):
        for h in range(ncol16):
            ref[r, pl.ds(16 * h, 16)] = _Z16()
        return 0
    lax.fori_loop(0, nrows, body, 0)


@functools.partial(
    pl.kernel,
    out_type=(
        jax.ShapeDtypeStruct((NSC, N, 16), jnp.float32),
        jax.ShapeDtypeStruct((NSC, B, 16), jnp.float32),
    ),
    mesh=_mesh,
    scratch_types=(
        pltpu.VMEM((K,), jnp.int32),
        pltpu.VMEM((K,), jnp.int32),
        pltpu.VMEM((16,), jnp.int32),
        pltpu.VMEM((K, 16), jnp.float32),
        pltpu.VMEM((QR, 16), jnp.float32),
        pltpu.VMEM_SHARED((N, 16), jnp.float32),
        pltpu.VMEM_SHARED((B, 16), jnp.float32),
        pltpu.SemaphoreType.DMA,
        pltpu.SemaphoreType.DMA,
    ),
)
def _sc_counts(dst_hbm, batch_hbm, degw_hbm, cntw_hbm, eb0, eb1, idx16,
               ones, zbuf, Dw, Cw, se0, se1):
    c = lax.axis_index("c")
    s = lax.axis_index("s")
    w = s * NSC + c

    one = jnp.ones((16,), jnp.float32)

    def fill_ones(r, _):
        ones[r, :] = one
        return 0
    lax.fori_loop(0, K, fill_ones, 0)

    _zero_rows(zbuf, QR, 1)
    nq = NQ // NW + jnp.where(w < NQ % NW, 1, 0)

    def zbody(q, _):
        pltpu.sync_copy(zbuf, Dw.at[pl.ds((w + q * NW) * QR, QR)])
        return 0
    lax.fori_loop(0, nq, zbody, 0)
    bper = B // NSUB
    pltpu.sync_copy(zbuf.at[pl.ds(0, bper)], Cw.at[pl.ds(s * bper, bper)])
    plsc.subcore_barrier()

    nch = NCHUNK // NW + jnp.where(w < NCHUNK % NW, 1, 0)

    def eload(j, eb, sem):
        ch = w + j * NW
        return pltpu.async_copy(dst_hbm.at[pl.ds(ch * K, K)], eb, sem)

    def ewait(eb, sem):
        pltpu.make_async_copy(dst_hbm.at[pl.ds(0, K)], eb, sem).wait()

    eload(0, eb0, se0)
    eload(1, eb1, se1)

    def ebody(m, _):
        for par, (ebA, seA) in enumerate(((eb0, se0), (eb1, se1))):
            j = 2 * m + par

            @pl.when(j < nch)
            def _():
                ewait(ebA, seA)
                pltpu.sync_copy(ones, Dw.at[ebA], add=True)

                @pl.when(j + 2 < nch)
                def _():
                    eload(j + 2, ebA, seA)
        return 0
    lax.fori_loop(0, (NCHUNK // NW + 2) // 2, ebody, 0)

    nbfull = N // K
    nb = nbfull // NW + jnp.where(w < nbfull % NW, 1, 0)

    def bbody(j, _):
        ch = w + j * NW
        pltpu.sync_copy(batch_hbm.at[pl.ds(ch * K, K)], eb0)
        pltpu.sync_copy(ones, Cw.at[eb0], add=True)
        return 0
    lax.fori_loop(0, nb, bbody, 0)

    @pl.when(w == nbfull % NW)
    def _():
        pltpu.sync_copy(batch_hbm.at[pl.ds(nbfull * K, N - nbfull * K)], idx16)
        pltpu.sync_copy(ones.at[pl.ds(0, N - nbfull * K)], Cw.at[idx16], add=True)

    plsc.subcore_barrier()

    def obody(q, _):
        r0 = (w + q * NW) * QR
        pltpu.sync_copy(Dw.at[pl.ds(r0, QR)], zbuf)
        pltpu.sync_copy(zbuf, degw_hbm.at[c, pl.ds(r0, QR)])
        return 0
    lax.fori_loop(0, nq, obody, 0)
    pltpu.sync_copy(Cw.at[pl.ds(s * bper, bper)], zbuf.at[pl.ds(0, bper)])
    pltpu.sync_copy(zbuf.at[pl.ds(0, bper)], cntw_hbm.at[c, pl.ds(s * bper, bper)])


KP = 80
NROW = E // KP
NGRP = NROW // 8
NQP = N // KP


@functools.partial(
    pl.kernel,
    out_type=jax.ShapeDtypeStruct((NSC, N, H), jnp.float32),
    mesh=_mesh,
    scratch_types=(
        tuple(pltpu.VMEM((KP, H), jnp.float32) for _ in range(4)),
        tuple(pltpu.VMEM((8, KP), jnp.int32) for _ in range(2)),
        tuple(pltpu.VMEM((8, KP), jnp.int32) for _ in range(2)),
        pltpu.VMEM_SHARED((N, H), jnp.float32),
        tuple(pltpu.SemaphoreType.DMA for _ in range(4)),
        tuple(pltpu.SemaphoreType.DMA for _ in range(4)),
        tuple(pltpu.SemaphoreType.DMA for _ in range(2)),
    ),
)
def _sc_prop(g_hbm, src2d_hbm, dst2d_hbm, out_hbm, rows, sgrp, dgrp, S,
             sg, sc, sgi):
    c = lax.axis_index("c")
    s = lax.axis_index("s")
    w = s * NSC + c

    _zero_rows(rows[0], KP, H // 16)
    nq = NQP // NW + jnp.where(w < NQP % NW, 1, 0)

    def zbody(q, _):
        pltpu.sync_copy(rows[0], S.at[pl.ds((w + q * NW) * KP, KP)])
        return 0
    lax.fori_loop(0, nq, zbody, 0)
    plsc.subcore_barrier()

    ngrp = NGRP // NW + jnp.where(w < NGRP % NW, 1, 0)

    def gidx_issue(t, p):
        gid = w + t * NW
        pltpu.async_copy(src2d_hbm.at[pl.ds(8 * gid, 8)], sgrp[p], sgi[p])
        pltpu.async_copy(dst2d_hbm.at[pl.ds(8 * gid, 8)], dgrp[p], sgi[p])

    def gidx_wait(p):
        pltpu.make_async_copy(src2d_hbm.at[pl.ds(0, 8)], sgrp[p], sgi[p]).wait()
        pltpu.make_async_copy(dst2d_hbm.at[pl.ds(0, 8)], dgrp[p], sgi[p]).wait()

    def gissue(p, u, q):
        pltpu.async_copy(g_hbm.at[sgrp[p].at[u]], rows[q], sg[q])

    def gwait(q):
        pltpu.make_async_copy(g_hbm.at[pl.ds(0, KP)], rows[q], sg[q]).wait()

    def scissue(p, u, q):
        pltpu.async_copy(rows[q], S.at[dgrp[p].at[u]], sc[q], add=True)

    def scwait(q):
        pltpu.make_async_copy(g_hbm.at[pl.ds(0, KP)], rows[q], sc[q]).wait()

    gidx_issue(0, 0)
    gidx_wait(0)
    gissue(0, 0, 0)

    def gbody(m, _):
        for gp in range(2):
            t = 2 * m + gp

            @pl.when(t < ngrp)
            def _():
                for u in range(8):
                    q = u % 4
                    if u < 3:
                        @pl.when(t > 0)
                        def _():
                            scwait((u + 1) % 4)
                    else:
                        scwait((u + 1) % 4)
                    if u == 0:
                        @pl.when(t + 1 < ngrp)
                        def _():
                            gidx_issue(t + 1, 1 - gp)
                    gwait(q)
                    scissue(gp, u, q)
                    if u < 7:
                        gissue(gp, u + 1, (u + 1) % 4)
                    else:
                        @pl.when(t + 1 < ngrp)
                        def _():
                            gidx_wait(1 - gp)
                            gissue(1 - gp, 0, 0)
        return 0
    lax.fori_loop(0, (NGRP // NW + 2) // 2, gbody, 0)

    scwait(1)
    scwait(2)
    scwait(3)

    plsc.subcore_barrier()

    def obody(q, _):
        r0 = (w + q * NW) * KP
        pltpu.sync_copy(S.at[pl.ds(r0, KP)], rows[0])
        pltpu.sync_copy(rows[0], out_hbm.at[c, pl.ds(r0, KP)])
        return 0
    lax.fori_loop(0, nq, obody, 0)


@functools.partial(
    pl.kernel,
    out_type=jax.ShapeDtypeStruct((B, H), jnp.float32),
    mesh=_mesh,
    scratch_types=(
        pltpu.VMEM((40,), jnp.int32),
        pltpu.VMEM((PCH, H), jnp.float32),
        pltpu.VMEM((SEGS_PER_W, H), jnp.float32),
    ),
)
def _sc_pool(h_hbm, starts_hbm, out_hbm, stv, buf, outbuf):
    c = lax.axis_index("c")
    s = lax.axis_index("s")
    w = s * NSC + c

    pltpu.sync_copy(starts_hbm.at[pl.ds(w * SEGS_PER_W, 24)],
                    stv.at[pl.ds(0, 24)])

    _zero_rows(outbuf, SEGS_PER_W, H // 16)

    a0 = stv[pl.ds(0, 16)][0]
    end = stv[pl.ds(16, 16)][0]
    base0 = pl.multiple_of((a0 // 8) * 8, 8)
    nchk = (end - base0 + PCH - 1) // PCH

    def ch_body(k2, _):
        cstart = pl.multiple_of(jnp.minimum(base0 + k2 * PCH, N - PCH), 8)
        pltpu.sync_copy(h_hbm.at[pl.ds(cstart, PCH)], buf)

        def seg_body(j, _2):
            sv = stv[pl.ds(j, 16)]
            lo = jnp.maximum(sv[0], cstart)
            hi = jnp.minimum(sv[1], cstart + PCH)
            nrows = jnp.maximum(hi - lo, 0)
            acc0 = tuple(outbuf[j, pl.ds(16 * h, 16)] for h in range(H // 16))

            def row_body(i, acc):
                r = lo - cstart + i
                return tuple(
                    jnp.maximum(acc[h], buf[r, pl.ds(16 * h, 16)])
                    for h in range(H // 16)
                )
            acc = lax.fori_loop(0, nrows, row_body, acc0)
            for h in range(H // 16):
                outbuf[j, pl.ds(16 * h, 16)] = acc[h]
            return 0
        lax.fori_loop(0, SEGS_PER_W, seg_body, 0)
        return 0
    lax.fori_loop(0, nchk, ch_body, 0)
    pltpu.sync_copy(outbuf, out_hbm.at[pl.ds(w * SEGS_PER_W, SEGS_PER_W)])


_BLK = 1000


def _prep_body(dw_ref, cw_ref, dinv_ref, starts_ref):
    deg = dw_ref[0, :, 0:1] + dw_ref[1, :, 0:1] + 1.0
    dinv_ref[...] = lax.rsqrt(deg)
    cnt = cw_ref[0, :, 0:1] + cw_ref[1, :, 0:1]
    row = lax.broadcasted_iota(jnp.int32, (B, B), 0)
    col = lax.broadcasted_iota(jnp.int32, (B, B), 1)
    tril = jnp.where(col < row, 1.0, 0.0)
    st = jnp.dot(tril, cnt, preferred_element_type=jnp.float32)
    starts_ref[pl.ds(0, B)] = st[:, 0].astype(jnp.int32)
    starts_ref[pl.ds(B, 8)] = jnp.full((8,), N, jnp.int32)


def _prep(degw, cntw):
    return pl.pallas_call(
        _prep_body,
        out_shape=(
            jax.ShapeDtypeStruct((N, 1), jnp.float32),
            jax.ShapeDtypeStruct((B + 8,), jnp.int32),
        ),
    )(degw, cntw)


def _mm_scale_body(x_ref, w_ref, dinv_ref, o_ref):
    o_ref[...] = jnp.dot(x_ref[...], w_ref[...],
                         preferred_element_type=jnp.float32) * dinv_ref[...]


def _mm_scale(x, W, dinv):
    return pl.pallas_call(
        _mm_scale_body,
        grid=(N // _BLK,),
        in_specs=[
            pl.BlockSpec((_BLK, H), lambda i: (i, 0)),
            pl.BlockSpec((H, H), lambda i: (0, 0)),
            pl.BlockSpec((_BLK, 1), lambda i: (i, 0)),
        ],
        out_specs=pl.BlockSpec((_BLK, H), lambda i: (i, 0)),
        out_shape=jax.ShapeDtypeStruct((N, H), jnp.float32),
    )(x, W, dinv)


def _layer_body(S_ref, g_ref, dinv_ref, b_ref, w_ref, o_ref):
    h = jnp.maximum(
        (S_ref[0] + S_ref[1] + g_ref[...]) * dinv_ref[...] + b_ref[...], 0.0)
    o_ref[...] = jnp.dot(h, w_ref[...],
                         preferred_element_type=jnp.float32) * dinv_ref[...]


def _layer(S, g, dinv, b, Wn):
    return pl.pallas_call(
        _layer_body,
        grid=(N // _BLK,),
        in_specs=[
            pl.BlockSpec((NSC, _BLK, H), lambda i: (0, i, 0)),
            pl.BlockSpec((_BLK, H), lambda i: (i, 0)),
            pl.BlockSpec((_BLK, 1), lambda i: (i, 0)),
            pl.BlockSpec((1, H), lambda i: (0, 0)),
            pl.BlockSpec((H, H), lambda i: (0, 0)),
        ],
        out_specs=pl.BlockSpec((_BLK, H), lambda i: (i, 0)),
        out_shape=jax.ShapeDtypeStruct((N, H), jnp.float32),
    )(S, g, dinv, b, Wn)


def _finalh_body(S_ref, g_ref, dinv_ref, b_ref, o_ref):
    o_ref[...] = jnp.maximum(
        (S_ref[0] + S_ref[1] + g_ref[...]) * dinv_ref[...] + b_ref[...], 0.0)


def _finalh(S, g, dinv, b):
    return pl.pallas_call(
        _finalh_body,
        grid=(N // _BLK,),
        in_specs=[
            pl.BlockSpec((NSC, _BLK, H), lambda i: (0, i, 0)),
            pl.BlockSpec((_BLK, H), lambda i: (i, 0)),
            pl.BlockSpec((_BLK, 1), lambda i: (i, 0)),
            pl.BlockSpec((1, H), lambda i: (0, 0)),
        ],
        out_specs=pl.BlockSpec((_BLK, H), lambda i: (i, 0)),
        out_shape=jax.ShapeDtypeStruct((N, H), jnp.float32),
    )(S, g, dinv, b)


def _head_body(p_ref, pr_ref, w1_ref, b1_ref, w2_ref, b2_ref, o_ref):
    z = jnp.dot(p_ref[...], w1_ref[0:H, :], preferred_element_type=jnp.float32)
    z = z + jnp.dot(pr_ref[...], w1_ref[H:, :],
                    preferred_element_type=jnp.float32)
    z = jnp.maximum(z + b1_ref[...], 0.0)
    o_ref[...] = jnp.dot(z, w2_ref[...],
                         preferred_element_type=jnp.float32) + b2_ref[...]


def _head(pooled, prot, fcW1, fcb1, fcW2p, fcb2):
    return pl.pallas_call(
        _head_body,
        out_shape=jax.ShapeDtypeStruct((B, H), jnp.float32),
    )(pooled, prot, fcW1, fcb1, fcW2p, fcb2)


@jax.jit
def kernel(x, edge_index, batch, prot_vec, W0, b0, W1, b1, W2, b2,
           fcW1, fcb1, fcW2, fcb2):
    src = edge_index[0]
    dst = edge_index[1]
    src2d = src.reshape(NROW, KP)
    dst2d = dst.reshape(NROW, KP)
    degw, cntw = _sc_counts(dst, batch)
    dinv, starts = _prep(degw, cntw)
    g0 = _mm_scale(x, W0, dinv)
    S0 = _sc_prop(g0, src2d, dst2d)
    g1 = _layer(S0, g0, dinv, b0.reshape(1, H), W1)
    S1 = _sc_prop(g1, src2d, dst2d)
    g2 = _layer(S1, g1, dinv, b1.reshape(1, H), W2)
    S2 = _sc_prop(g2, src2d, dst2d)
    h3 = _finalh(S2, g2, dinv, b2.reshape(1, H))
    pooled = _sc_pool(h3, starts)
    fcW2p = jnp.pad(fcW2, ((0, 0), (0, H - 1)))
    res = _head(pooled, prot_vec, fcW1, fcb1.reshape(1, 256),
                fcW2p, fcb2.reshape(1, 1))
    return res[:, :1]


# speedup vs baseline: 1.2091x; 1.2091x over previous
import functools

import jax
import jax.numpy as jnp
from jax import lax
from jax.experimental import pallas as pl
from jax.experimental.pallas import tpu as pltpu
from jax.experimental.pallas import tpu_sc as plsc

N = 10000
E = 640000
B = 512
H = 128
PROT = 128

NSC = 2
NSUB = 16
NW = NSC * NSUB
K = 128
NCHUNK = E // K
QR = 80
NQ = N // QR
SEGS_PER_W = B // NW
PCH = 256

_mesh = plsc.VectorSubcoreMesh(core_axis_name="c", subcore_axis_name="s")

_Z16 = functools.partial(jnp.zeros, (16,), jnp.float32)


def _zero_rows(ref, nrows, ncol16):
    def body(r, _):
        for h in range(ncol16):
            ref[r, pl.ds(16 * h, 16)] = _Z16()
        return 0
    lax.fori_loop(0, nrows, body, 0)


@functools.partial(
    pl.kernel,
    out_type=(
        jax.ShapeDtypeStruct((NSC, N, 16), jnp.float32),
        jax.ShapeDtypeStruct((NSC, B, 16), jnp.float32),
    ),
    mesh=_mesh,
    scratch_types=(
        pltpu.VMEM((K,), jnp.int32),
        pltpu.VMEM((K,), jnp.int32),
        pltpu.VMEM((16,), jnp.int32),
        pltpu.VMEM((K, 16), jnp.float32),
        pltpu.VMEM((QR, 16), jnp.float32),
        pltpu.VMEM_SHARED((N, 16), jnp.float32),
        pltpu.VMEM_SHARED((B, 16), jnp.float32),
        pltpu.SemaphoreType.DMA,
        pltpu.SemaphoreType.DMA,
    ),
)
def _sc_counts(dst_hbm, batch_hbm, degw_hbm, cntw_hbm, eb0, eb1, idx16,
               ones, zbuf, Dw, Cw, se0, se1):
    c = lax.axis_index("c")
    s = lax.axis_index("s")
    w = s * NSC + c

    one = jnp.ones((16,), jnp.float32)

    def fill_ones(r, _):
        ones[r, :] = one
        return 0
    lax.fori_loop(0, K, fill_ones, 0)

    _zero_rows(zbuf, QR, 1)
    nq = NQ // NW + jnp.where(w < NQ % NW, 1, 0)

    def zbody(q, _):
        pltpu.sync_copy(zbuf, Dw.at[pl.ds((w + q * NW) * QR, QR)])
        return 0
    lax.fori_loop(0, nq, zbody, 0)
    bper = B // NSUB
    pltpu.sync_copy(zbuf.at[pl.ds(0, bper)], Cw.at[pl.ds(s * bper, bper)])
    plsc.subcore_barrier()

    nch = NCHUNK // NW + jnp.where(w < NCHUNK % NW, 1, 0)

    def eload(j, eb, sem):
        ch = w + j * NW
        return pltpu.async_copy(dst_hbm.at[pl.ds(ch * K, K)], eb, sem)

    def ewait(eb, sem):
        pltpu.make_async_copy(dst_hbm.at[pl.ds(0, K)], eb, sem).wait()

    eload(0, eb0, se0)
    eload(1, eb1, se1)

    def ebody(m, _):
        for par, (ebA, seA) in enumerate(((eb0, se0), (eb1, se1))):
            j = 2 * m + par

            @pl.when(j < nch)
            def _():
                ewait(ebA, seA)
                pltpu.sync_copy(ones, Dw.at[ebA], add=True)

                @pl.when(j + 2 < nch)
                def _():
                    eload(j + 2, ebA, seA)
        return 0
    lax.fori_loop(0, (NCHUNK // NW + 2) // 2, ebody, 0)

    nbfull = N // K
    nb = nbfull // NW + jnp.where(w < nbfull % NW, 1, 0)

    def bbody(j, _):
        ch = w + j * NW
        pltpu.sync_copy(batch_hbm.at[pl.ds(ch * K, K)], eb0)
        pltpu.sync_copy(ones, Cw.at[eb0], add=True)
        return 0
    lax.fori_loop(0, nb, bbody, 0)

    @pl.when(w == nbfull % NW)
    def _():
        pltpu.sync_copy(batch_hbm.at[pl.ds(nbfull * K, N - nbfull * K)], idx16)
        pltpu.sync_copy(ones.at[pl.ds(0, N - nbfull * K)], Cw.at[idx16], add=True)

    plsc.subcore_barrier()

    def obody(q, _):
        r0 = (w + q * NW) * QR
        pltpu.sync_copy(Dw.at[pl.ds(r0, QR)], zbuf)
        pltpu.sync_copy(zbuf, degw_hbm.at[c, pl.ds(r0, QR)])
        return 0
    lax.fori_loop(0, nq, obody, 0)
    pltpu.sync_copy(Cw.at[pl.ds(s * bper, bper)], zbuf.at[pl.ds(0, bper)])
    pltpu.sync_copy(zbuf.at[pl.ds(0, bper)], cntw_hbm.at[c, pl.ds(s * bper, bper)])


@functools.partial(
    pl.kernel,
    out_type=jax.ShapeDtypeStruct((NSC, N, H), jnp.float32),
    mesh=_mesh,
    scratch_types=(
        pltpu.VMEM((K,), jnp.int32),
        pltpu.VMEM((K,), jnp.int32),
        pltpu.VMEM((K,), jnp.int32),
        pltpu.VMEM((K,), jnp.int32),
        pltpu.VMEM((K, H), jnp.float32),
        pltpu.VMEM((K, H), jnp.float32),
        pltpu.VMEM((QR, H), jnp.float32),
        pltpu.VMEM_SHARED((N, H), jnp.float32),
        pltpu.SemaphoreType.DMA,
        pltpu.SemaphoreType.DMA,
        pltpu.SemaphoreType.DMA,
        pltpu.SemaphoreType.DMA,
        pltpu.SemaphoreType.DMA,
        pltpu.SemaphoreType.DMA,
    ),
)
def _sc_prop(g_hbm, src_hbm, dst_hbm, out_hbm, si0, si1, di0, di1,
             rows0, rows1, zbuf, S, ss0, ss1, sd0, sd1, sg0, sg1):
    c = lax.axis_index("c")
    s = lax.axis_index("s")
    w = s * NSC + c

    _zero_rows(zbuf, QR, H // 16)
    nq = NQ // NW + jnp.where(w < NQ % NW, 1, 0)

    def zbody(q, _):
        pltpu.sync_copy(zbuf, S.at[pl.ds((w + q * NW) * QR, QR)])
        return 0
    lax.fori_loop(0, nq, zbody, 0)
    plsc.subcore_barrier()

    nch = NCHUNK // NW + jnp.where(w < NCHUNK % NW, 1, 0)

    def iload(j, si, di, ss, sd):
        ch = w + j * NW
        pltpu.async_copy(src_hbm.at[pl.ds(ch * K, K)], si, ss)
        pltpu.async_copy(dst_hbm.at[pl.ds(ch * K, K)], di, sd)

    def iwait(si, di, ss, sd):
        pltpu.make_async_copy(src_hbm.at[pl.ds(0, K)], si, ss).wait()
        pltpu.make_async_copy(dst_hbm.at[pl.ds(0, K)], di, sd).wait()

    def gwait(rows, sg):
        pltpu.make_async_copy(g_hbm.at[pl.ds(0, K)], rows, sg).wait()

    bufs = ((si0, di0, rows0, ss0, sd0, sg0), (si1, di1, rows1, ss1, sd1, sg1))

    iload(0, si0, di0, ss0, sd0)
    iwait(si0, di0, ss0, sd0)
    pltpu.async_copy(g_hbm.at[si0], rows0, sg0)
    iload(1, si1, di1, ss1, sd1)

    def ebody(m, _):
        for par in range(2):
            j = 2 * m + par
            siA, diA, rowsA, ssA, sdA, sgA = bufs[par]
            siB, diB, rowsB, ssB, sdB, sgB = bufs[1 - par]

            @pl.when(j < nch)
            def _():
                gwait(rowsA, sgA)

                @pl.when(j + 1 < nch)
                def _():
                    iwait(siB, diB, ssB, sdB)
                    pltpu.async_copy(g_hbm.at[siB], rowsB, sgB)

                pltpu.sync_copy(rowsA, S.at[diA], add=True)

                @pl.when(j + 2 < nch)
                def _():
                    iload(j + 2, siA, diA, ssA, sdA)
        return 0
    lax.fori_loop(0, (NCHUNK // NW + 2) // 2, ebody, 0)

    plsc.subcore_barrier()

    def obody(q, _):
        r0 = (w + q * NW) * QR
        pltpu.sync_copy(S.at[pl.ds(r0, QR)], zbuf)
        pltpu.sync_copy(zbuf, out_hbm.at[c, pl.ds(r0, QR)])
        return 0
    lax.fori_loop(0, nq, obody, 0)


@functools.partial(
    pl.kernel,
    out_type=jax.ShapeDtypeStruct((B, H), jnp.float32),
    mesh=_mesh,
    scratch_types=(
        pltpu.VMEM((40,), jnp.int32),
        pltpu.VMEM((PCH, H), jnp.float32),
        pltpu.VMEM((SEGS_PER_W, H), jnp.float32),
    ),
)
def _sc_pool(h_hbm, starts_hbm, out_hbm, stv, buf, outbuf):
    c = lax.axis_index("c")
    s = lax.axis_index("s")
    w = s * NSC + c

    pltpu.sync_copy(starts_hbm.at[pl.ds(w * SEGS_PER_W, 24)],
                    stv.at[pl.ds(0, 24)])

    _zero_rows(outbuf, SEGS_PER_W, H // 16)

    a0 = stv[pl.ds(0, 16)][0]
    end = stv[pl.ds(16, 16)][0]
    base0 = pl.multiple_of((a0 // 8) * 8, 8)
    nchk = (end - base0 + PCH - 1) // PCH

    def ch_body(k2, _):
        cstart = pl.multiple_of(jnp.minimum(base0 + k2 * PCH, N - PCH), 8)
        pltpu.sync_copy(h_hbm.at[pl.ds(cstart, PCH)], buf)

        def seg_body(j, _2):
            sv = stv[pl.ds(j, 16)]
            lo = jnp.maximum(sv[0], cstart)
            hi = jnp.minimum(sv[1], cstart + PCH)
            nrows = jnp.maximum(hi - lo, 0)
            acc0 = tuple(outbuf[j, pl.ds(16 * h, 16)] for h in range(H // 16))

            def row_body(i, acc):
                r = lo - cstart + i
                return tuple(
                    jnp.maximum(acc[h], buf[r, pl.ds(16 * h, 16)])
                    for h in range(H // 16)
                )
            acc = lax.fori_loop(0, nrows, row_body, acc0)
            for h in range(H // 16):
                outbuf[j, pl.ds(16 * h, 16)] = acc[h]
            return 0
        lax.fori_loop(0, SEGS_PER_W, seg_body, 0)
        return 0
    lax.fori_loop(0, nchk, ch_body, 0)
    pltpu.sync_copy(outbuf, out_hbm.at[pl.ds(w * SEGS_PER_W, SEGS_PER_W)])


_BLK = 1000


def _prep_body(dw_ref, cw_ref, dinv_ref, starts_ref):
    deg = dw_ref[0, :, 0:1] + dw_ref[1, :, 0:1] + 1.0
    dinv_ref[...] = lax.rsqrt(deg)
    cnt = cw_ref[0, :, 0:1] + cw_ref[1, :, 0:1]
    row = lax.broadcasted_iota(jnp.int32, (B, B), 0)
    col = lax.broadcasted_iota(jnp.int32, (B, B), 1)
    tril = jnp.where(col < row, 1.0, 0.0)
    st = jnp.dot(tril, cnt, preferred_element_type=jnp.float32)
    starts_ref[pl.ds(0, B)] = st[:, 0].astype(jnp.int32)
    starts_ref[pl.ds(B, 8)] = jnp.full((8,), N, jnp.int32)


def _prep(degw, cntw):
    return pl.pallas_call(
        _prep_body,
        out_shape=(
            jax.ShapeDtypeStruct((N, 1), jnp.float32),
            jax.ShapeDtypeStruct((B + 8,), jnp.int32),
        ),
    )(degw, cntw)


def _mm_scale_body(x_ref, w_ref, dinv_ref, o_ref):
    o_ref[...] = jnp.dot(x_ref[...], w_ref[...],
                         preferred_element_type=jnp.float32) * dinv_ref[...]


def _mm_scale(x, W, dinv):
    return pl.pallas_call(
        _mm_scale_body,
        grid=(N // _BLK,),
        in_specs=[
            pl.BlockSpec((_BLK, H), lambda i: (i, 0)),
            pl.BlockSpec((H, H), lambda i: (0, 0)),
            pl.BlockSpec((_BLK, 1), lambda i: (i, 0)),
        ],
        out_specs=pl.BlockSpec((_BLK, H), lambda i: (i, 0)),
        out_shape=jax.ShapeDtypeStruct((N, H), jnp.float32),
    )(x, W, dinv)


def _layer_body(S_ref, g_ref, dinv_ref, b_ref, w_ref, o_ref):
    h = jnp.maximum(
        (S_ref[0] + S_ref[1] + g_ref[...]) * dinv_ref[...] + b_ref[...], 0.0)
    o_ref[...] = jnp.dot(h, w_ref[...],
                         preferred_element_type=jnp.float32) * dinv_ref[...]


def _layer(S, g, dinv, b, Wn):
    return pl.pallas_call(
        _layer_body,
        grid=(N // _BLK,),
        in_specs=[
            pl.BlockSpec((NSC, _BLK, H), lambda i: (0, i, 0)),
            pl.BlockSpec((_BLK, H), lambda i: (i, 0)),
            pl.BlockSpec((_BLK, 1), lambda i: (i, 0)),
            pl.BlockSpec((1, H), lambda i: (0, 0)),
            pl.BlockSpec((H, H), lambda i: (0, 0)),
        ],
        out_specs=pl.BlockSpec((_BLK, H), lambda i: (i, 0)),
        out_shape=jax.ShapeDtypeStruct((N, H), jnp.float32),
    )(S, g, dinv, b, Wn)


def _finalh_body(S_ref, g_ref, dinv_ref, b_ref, o_ref):
    o_ref[...] = jnp.maximum(
        (S_ref[0] + S_ref[1] + g_ref[...]) * dinv_ref[...] + b_ref[...], 0.0)


def _finalh(S, g, dinv, b):
    return pl.pallas_call(
        _finalh_body,
        grid=(N // _BLK,),
        in_specs=[
            pl.BlockSpec((NSC, _BLK, H), lambda i: (0, i, 0)),
            pl.BlockSpec((_BLK, H), lambda i: (i, 0)),
            pl.BlockSpec((_BLK, 1), lambda i: (i, 0)),
            pl.BlockSpec((1, H), lambda i: (0, 0)),
        ],
        out_specs=pl.BlockSpec((_BLK, H), lambda i: (i, 0)),
        out_shape=jax.ShapeDtypeStruct((N, H), jnp.float32),
    )(S, g, dinv, b)


def _head_body(p_ref, pr_ref, w1_ref, b1_ref, w2_ref, b2_ref, o_ref):
    z = jnp.dot(p_ref[...], w1_ref[0:H, :], preferred_element_type=jnp.float32)
    z = z + jnp.dot(pr_ref[...], w1_ref[H:, :],
                    preferred_element_type=jnp.float32)
    z = jnp.maximum(z + b1_ref[...], 0.0)
    o_ref[...] = jnp.dot(z, w2_ref[...],
                         preferred_element_type=jnp.float32) + b2_ref[...]


def _head(pooled, prot, fcW1, fcb1, fcW2p, fcb2):
    return pl.pallas_call(
        _head_body,
        out_shape=jax.ShapeDtypeStruct((B, H), jnp.float32),
    )(pooled, prot, fcW1, fcb1, fcW2p, fcb2)


@jax.jit
def kernel(x, edge_index, batch, prot_vec, W0, b0, W1, b1, W2, b2,
           fcW1, fcb1, fcW2, fcb2):
    src = edge_index[0]
    dst = edge_index[1]
    degw, cntw = _sc_counts(dst, batch)
    dinv, starts = _prep(degw, cntw)
    g0 = _mm_scale(x, W0, dinv)
    S0 = _sc_prop(g0, src, dst)
    g1 = _layer(S0, g0, dinv, b0.reshape(1, H), W1)
    S1 = _sc_prop(g1, src, dst)
    g2 = _layer(S1, g1, dinv, b1.reshape(1, H), W2)
    S2 = _sc_prop(g2, src, dst)
    h3 = _finalh(S2, g2, dinv, b2.reshape(1, H))
    pooled = _sc_pool(h3, starts)
    fcW2p = jnp.pad(fcW2, ((0, 0), (0, H - 1)))
    res = _head(pooled, prot_vec, fcW1, fcb1.reshape(1, 256),
                fcW2p, fcb2.reshape(1, 1))
    return res[:, :1]


# speedup vs baseline: 1.2966x; 1.0724x over previous
import functools

import jax
import jax.numpy as jnp
from jax import lax
from jax.experimental import pallas as pl
from jax.experimental.pallas import tpu as pltpu
from jax.experimental.pallas import tpu_sc as plsc

N = 10000
E = 640000
B = 512
H = 128
PROT = 128

NSC = 2
NSUB = 16
NW = NSC * NSUB
K = 128
NCHUNK = E // K
QR = 80
NQ = N // QR
SEGS_PER_W = B // NW
PCH = 384

_mesh = plsc.VectorSubcoreMesh(core_axis_name="c", subcore_axis_name="s")

_Z16 = functools.partial(jnp.zeros, (16,), jnp.float32)


def _zero_rows(ref, nrows, ncol16):
    def body(r, _):
        for h in range(ncol16):
            ref[r, pl.ds(16 * h, 16)] = _Z16()
        return 0
    lax.fori_loop(0, nrows, body, 0)


@functools.partial(
    pl.kernel,
    out_type=(
        jax.ShapeDtypeStruct((NSC, N, 16), jnp.float32),
        jax.ShapeDtypeStruct((NSC, B, 16), jnp.float32),
    ),
    mesh=_mesh,
    scratch_types=(
        pltpu.VMEM((K,), jnp.int32),
        pltpu.VMEM((K,), jnp.int32),
        pltpu.VMEM((16,), jnp.int32),
        pltpu.VMEM((K, 16), jnp.float32),
        pltpu.VMEM((QR, 16), jnp.float32),
        pltpu.VMEM_SHARED((N, 16), jnp.float32),
        pltpu.VMEM_SHARED((B, 16), jnp.float32),
        pltpu.SemaphoreType.DMA,
        pltpu.SemaphoreType.DMA,
    ),
)
def _sc_counts(dst_hbm, batch_hbm, degw_hbm, cntw_hbm, eb0, eb1, idx16,
               ones, zbuf, Dw, Cw, se0, se1):
    c = lax.axis_index("c")
    s = lax.axis_index("s")
    w = s * NSC + c

    one = jnp.ones((16,), jnp.float32)

    def fill_ones(r, _):
        ones[r, :] = one
        return 0
    lax.fori_loop(0, K, fill_ones, 0)

    _zero_rows(zbuf, QR, 1)
    nq = NQ // NW + jnp.where(w < NQ % NW, 1, 0)

    def zbody(q, _):
        pltpu.sync_copy(zbuf, Dw.at[pl.ds((w + q * NW) * QR, QR)])
        return 0
    lax.fori_loop(0, nq, zbody, 0)
    bper = B // NSUB
    pltpu.sync_copy(zbuf.at[pl.ds(0, bper)], Cw.at[pl.ds(s * bper, bper)])
    plsc.subcore_barrier()

    nch = NCHUNK // NW + jnp.where(w < NCHUNK % NW, 1, 0)

    def eload(j, eb, sem):
        ch = w + j * NW
        return pltpu.async_copy(dst_hbm.at[pl.ds(ch * K, K)], eb, sem)

    def ewait(eb, sem):
        pltpu.make_async_copy(dst_hbm.at[pl.ds(0, K)], eb, sem).wait()

    eload(0, eb0, se0)
    eload(1, eb1, se1)

    def ebody(m, _):
        for par, (ebA, seA) in enumerate(((eb0, se0), (eb1, se1))):
            j = 2 * m + par

            @pl.when(j < nch)
            def _():
                ewait(ebA, seA)
                pltpu.sync_copy(ones, Dw.at[ebA], add=True)

                @pl.when(j + 2 < nch)
                def _():
                    eload(j + 2, ebA, seA)
        return 0
    lax.fori_loop(0, (NCHUNK // NW + 2) // 2, ebody, 0)

    nbfull = N // K
    nb = nbfull // NW + jnp.where(w < nbfull % NW, 1, 0)

    def bbody(j, _):
        ch = w + j * NW
        pltpu.sync_copy(batch_hbm.at[pl.ds(ch * K, K)], eb0)
        pltpu.sync_copy(ones, Cw.at[eb0], add=True)
        return 0
    lax.fori_loop(0, nb, bbody, 0)

    @pl.when(w == nbfull % NW)
    def _():
        pltpu.sync_copy(batch_hbm.at[pl.ds(nbfull * K, N - nbfull * K)], idx16)
        pltpu.sync_copy(ones.at[pl.ds(0, N - nbfull * K)], Cw.at[idx16], add=True)

    plsc.subcore_barrier()

    def obody(q, _):
        r0 = (w + q * NW) * QR
        pltpu.sync_copy(Dw.at[pl.ds(r0, QR)], zbuf)
        pltpu.sync_copy(zbuf, degw_hbm.at[c, pl.ds(r0, QR)])
        return 0
    lax.fori_loop(0, nq, obody, 0)
    pltpu.sync_copy(Cw.at[pl.ds(s * bper, bper)], zbuf.at[pl.ds(0, bper)])
    pltpu.sync_copy(zbuf.at[pl.ds(0, bper)], cntw_hbm.at[c, pl.ds(s * bper, bper)])


@functools.partial(
    pl.kernel,
    out_type=jax.ShapeDtypeStruct((NSC, N, H), jnp.float32),
    mesh=_mesh,
    scratch_types=(
        pltpu.VMEM((K,), jnp.int32),
        pltpu.VMEM((K,), jnp.int32),
        pltpu.VMEM((K,), jnp.int32),
        pltpu.VMEM((K,), jnp.int32),
        pltpu.VMEM((K, H), jnp.float32),
        pltpu.VMEM((K, H), jnp.float32),
        pltpu.VMEM((QR, H), jnp.float32),
        pltpu.VMEM_SHARED((N, H), jnp.float32),
        pltpu.SemaphoreType.DMA,
        pltpu.SemaphoreType.DMA,
        pltpu.SemaphoreType.DMA,
        pltpu.SemaphoreType.DMA,
        pltpu.SemaphoreType.DMA,
        pltpu.SemaphoreType.DMA,
    ),
)
def _sc_prop(g_hbm, src_hbm, dst_hbm, out_hbm, si0, si1, di0, di1,
             rows0, rows1, zbuf, S, ss0, ss1, sd0, sd1, sg0, sg1):
    c = lax.axis_index("c")
    s = lax.axis_index("s")
    w = s * NSC + c

    _zero_rows(zbuf, QR, H // 16)
    nq = NQ // NW + jnp.where(w < NQ % NW, 1, 0)

    def zbody(q, _):
        pltpu.sync_copy(zbuf, S.at[pl.ds((w + q * NW) * QR, QR)])
        return 0
    lax.fori_loop(0, nq, zbody, 0)
    plsc.subcore_barrier()

    nch = NCHUNK // NW + jnp.where(w < NCHUNK % NW, 1, 0)

    def iload(j, si, di, ss, sd):
        ch = w + j * NW
        pltpu.async_copy(src_hbm.at[pl.ds(ch * K, K)], si, ss)
        pltpu.async_copy(dst_hbm.at[pl.ds(ch * K, K)], di, sd)

    def iwait(si, di, ss, sd):
        pltpu.make_async_copy(src_hbm.at[pl.ds(0, K)], si, ss).wait()
        pltpu.make_async_copy(dst_hbm.at[pl.ds(0, K)], di, sd).wait()

    def gwait(rows, sg):
        pltpu.make_async_copy(g_hbm.at[pl.ds(0, K)], rows, sg).wait()

    bufs = ((si0, di0, rows0, ss0, sd0, sg0), (si1, di1, rows1, ss1, sd1, sg1))

    iload(0, si0, di0, ss0, sd0)
    iwait(si0, di0, ss0, sd0)
    pltpu.async_copy(g_hbm.at[si0], rows0, sg0)
    iload(1, si1, di1, ss1, sd1)

    def ebody(m, _):
        for par in range(2):
            j = 2 * m + par
            siA, diA, rowsA, ssA, sdA, sgA = bufs[par]
            siB, diB, rowsB, ssB, sdB, sgB = bufs[1 - par]

            @pl.when(j < nch)
            def _():
                gwait(rowsA, sgA)

                @pl.when(j + 1 < nch)
                def _():
                    iwait(siB, diB, ssB, sdB)
                    pltpu.async_copy(g_hbm.at[siB], rowsB, sgB)

                pltpu.sync_copy(rowsA, S.at[diA], add=True)

                @pl.when(j + 2 < nch)
                def _():
                    iload(j + 2, siA, diA, ssA, sdA)
        return 0
    lax.fori_loop(0, (NCHUNK // NW + 2) // 2, ebody, 0)

    plsc.subcore_barrier()

    def obody(q, _):
        r0 = (w + q * NW) * QR
        pltpu.sync_copy(S.at[pl.ds(r0, QR)], zbuf)
        pltpu.sync_copy(zbuf, out_hbm.at[c, pl.ds(r0, QR)])
        return 0
    lax.fori_loop(0, nq, obody, 0)


@functools.partial(
    pl.kernel,
    out_type=jax.ShapeDtypeStruct((B, H), jnp.float32),
    mesh=_mesh,
    scratch_types=(
        pltpu.VMEM((40,), jnp.int32),
        pltpu.VMEM((PCH, H), jnp.float32),
        pltpu.VMEM((PCH, H), jnp.float32),
        pltpu.VMEM((SEGS_PER_W, H), jnp.float32),
        pltpu.SemaphoreType.DMA,
        pltpu.SemaphoreType.DMA,
    ),
)
def _sc_pool(h_hbm, starts_hbm, out_hbm, stv, buf0, buf1, outbuf, sb0, sb1):
    c = lax.axis_index("c")
    s = lax.axis_index("s")
    w = s * NSC + c

    pltpu.sync_copy(starts_hbm.at[pl.ds(w * SEGS_PER_W, 24)],
                    stv.at[pl.ds(0, 24)])

    _zero_rows(outbuf, SEGS_PER_W, H // 16)

    a0 = stv[pl.ds(0, 16)][0]
    end = stv[pl.ds(16, 16)][0]
    base0 = pl.multiple_of((a0 // 8) * 8, 8)
    nchk = (end - base0 + PCH - 1) // PCH
    bufs = ((buf0, sb0), (buf1, sb1))

    def cstart_of(k2):
        return pl.multiple_of(jnp.minimum(base0 + k2 * PCH, N - PCH), 8)

    def chunk_issue(k2, p):
        pltpu.async_copy(h_hbm.at[pl.ds(cstart_of(k2), PCH)], bufs[p][0],
                         bufs[p][1])

    def chunk_wait(p):
        pltpu.make_async_copy(h_hbm.at[pl.ds(0, PCH)], bufs[p][0],
                              bufs[p][1]).wait()

    @pl.when(nchk > 0)
    def _():
        chunk_issue(0, 0)

    @pl.when(nchk > 1)
    def _():
        chunk_issue(1, 1)

    def ch_body(m, _):
        for p in range(2):
            k2 = 2 * m + p

            @pl.when(k2 < nchk)
            def _():
                chunk_wait(p)
                buf = bufs[p][0]
                cstart = cstart_of(k2)

                def seg_body(j, _2):
                    sv = stv[pl.ds(j, 16)]
                    lo = jnp.maximum(sv[0], cstart)
                    hi = jnp.minimum(sv[1], cstart + PCH)
                    nrows = jnp.maximum(hi - lo, 0)
                    r0 = lo - cstart
                    acc0 = tuple(outbuf[j, pl.ds(16 * h, 16)]
                                 for h in range(H // 16))

                    def row4_body(i, acc):
                        r = r0 + 4 * i
                        for rr in range(4):
                            acc = tuple(
                                jnp.maximum(acc[h],
                                            buf[r + rr, pl.ds(16 * h, 16)])
                                for h in range(H // 16)
                            )
                        return acc

                    acc = lax.fori_loop(0, nrows // 4, row4_body, acc0)

                    def row1_body(i, acc):
                        r = r0 + (nrows // 4) * 4 + i
                        return tuple(
                            jnp.maximum(acc[h], buf[r, pl.ds(16 * h, 16)])
                            for h in range(H // 16)
                        )
                    acc = lax.fori_loop(0, nrows % 4, row1_body, acc)
                    for h in range(H // 16):
                        outbuf[j, pl.ds(16 * h, 16)] = acc[h]
                    return 0
                lax.fori_loop(0, SEGS_PER_W, seg_body, 0)

                @pl.when(k2 + 2 < nchk)
                def _():
                    chunk_issue(k2 + 2, p)
        return 0
    lax.fori_loop(0, (nchk + 1) // 2, ch_body, 0)
    pltpu.sync_copy(outbuf, out_hbm.at[pl.ds(w * SEGS_PER_W, SEGS_PER_W)])


_BLK = 1000


def _prep_body(dw_ref, cw_ref, dinv_ref, starts_ref):
    deg = dw_ref[0, :, 0:1] + dw_ref[1, :, 0:1] + 1.0
    dinv_ref[...] = lax.rsqrt(deg)
    cnt = cw_ref[0, :, 0:1] + cw_ref[1, :, 0:1]
    row = lax.broadcasted_iota(jnp.int32, (B, B), 0)
    col = lax.broadcasted_iota(jnp.int32, (B, B), 1)
    tril = jnp.where(col < row, 1.0, 0.0)
    st = jnp.dot(tril, cnt, preferred_element_type=jnp.float32)
    starts_ref[pl.ds(0, B)] = st[:, 0].astype(jnp.int32)
    starts_ref[pl.ds(B, 8)] = jnp.full((8,), N, jnp.int32)


def _prep(degw, cntw):
    return pl.pallas_call(
        _prep_body,
        out_shape=(
            jax.ShapeDtypeStruct((N, 1), jnp.float32),
            jax.ShapeDtypeStruct((B + 8,), jnp.int32),
        ),
    )(degw, cntw)


def _mm_scale_body(x_ref, w_ref, dinv_ref, o_ref):
    o_ref[...] = jnp.dot(x_ref[...], w_ref[...],
                         preferred_element_type=jnp.float32) * dinv_ref[...]


def _mm_scale(x, W, dinv):
    return pl.pallas_call(
        _mm_scale_body,
        grid=(N // _BLK,),
        in_specs=[
            pl.BlockSpec((_BLK, H), lambda i: (i, 0)),
            pl.BlockSpec((H, H), lambda i: (0, 0)),
            pl.BlockSpec((_BLK, 1), lambda i: (i, 0)),
        ],
        out_specs=pl.BlockSpec((_BLK, H), lambda i: (i, 0)),
        out_shape=jax.ShapeDtypeStruct((N, H), jnp.float32),
    )(x, W, dinv)


def _layer_body(S_ref, g_ref, dinv_ref, b_ref, w_ref, o_ref):
    h = jnp.maximum(
        (S_ref[0] + S_ref[1] + g_ref[...]) * dinv_ref[...] + b_ref[...], 0.0)
    o_ref[...] = jnp.dot(h, w_ref[...],
                         preferred_element_type=jnp.float32) * dinv_ref[...]


def _layer(S, g, dinv, b, Wn):
    return pl.pallas_call(
        _layer_body,
        grid=(N // _BLK,),
        in_specs=[
            pl.BlockSpec((NSC, _BLK, H), lambda i: (0, i, 0)),
            pl.BlockSpec((_BLK, H), lambda i: (i, 0)),
            pl.BlockSpec((_BLK, 1), lambda i: (i, 0)),
            pl.BlockSpec((1, H), lambda i: (0, 0)),
            pl.BlockSpec((H, H), lambda i: (0, 0)),
        ],
        out_specs=pl.BlockSpec((_BLK, H), lambda i: (i, 0)),
        out_shape=jax.ShapeDtypeStruct((N, H), jnp.float32),
    )(S, g, dinv, b, Wn)


def _finalh_body(S_ref, g_ref, dinv_ref, b_ref, o_ref):
    o_ref[...] = jnp.maximum(
        (S_ref[0] + S_ref[1] + g_ref[...]) * dinv_ref[...] + b_ref[...], 0.0)


def _finalh(S, g, dinv, b):
    return pl.pallas_call(
        _finalh_body,
        grid=(N // _BLK,),
        in_specs=[
            pl.BlockSpec((NSC, _BLK, H), lambda i: (0, i, 0)),
            pl.BlockSpec((_BLK, H), lambda i: (i, 0)),
            pl.BlockSpec((_BLK, 1), lambda i: (i, 0)),
            pl.BlockSpec((1, H), lambda i: (0, 0)),
        ],
        out_specs=pl.BlockSpec((_BLK, H), lambda i: (i, 0)),
        out_shape=jax.ShapeDtypeStruct((N, H), jnp.float32),
    )(S, g, dinv, b)


def _head_body(p_ref, pr_ref, w1_ref, b1_ref, w2_ref, b2_ref, o_ref):
    z = jnp.dot(p_ref[...], w1_ref[0:H, :], preferred_element_type=jnp.float32)
    z = z + jnp.dot(pr_ref[...], w1_ref[H:, :],
                    preferred_element_type=jnp.float32)
    z = jnp.maximum(z + b1_ref[...], 0.0)
    o_ref[...] = jnp.dot(z, w2_ref[...],
                         preferred_element_type=jnp.float32) + b2_ref[...]


def _head(pooled, prot, fcW1, fcb1, fcW2p, fcb2):
    return pl.pallas_call(
        _head_body,
        out_shape=jax.ShapeDtypeStruct((B, H), jnp.float32),
    )(pooled, prot, fcW1, fcb1, fcW2p, fcb2)


@jax.jit
def kernel(x, edge_index, batch, prot_vec, W0, b0, W1, b1, W2, b2,
           fcW1, fcb1, fcW2, fcb2):
    src = edge_index[0]
    dst = edge_index[1]
    degw, cntw = _sc_counts(dst, batch)
    dinv, starts = _prep(degw, cntw)
    g0 = _mm_scale(x, W0, dinv)
    S0 = _sc_prop(g0, src, dst)
    g1 = _layer(S0, g0, dinv, b0.reshape(1, H), W1)
    S1 = _sc_prop(g1, src, dst)
    g2 = _layer(S1, g1, dinv, b1.reshape(1, H), W2)
    S2 = _sc_prop(g2, src, dst)
    h3 = _finalh(S2, g2, dinv, b2.reshape(1, H))
    pooled = _sc_pool(h3, starts)
    fcW2p = jnp.pad(fcW2, ((0, 0), (0, H - 1)))
    res = _head(pooled, prot_vec, fcW1, fcb1.reshape(1, 256),
                fcW2p, fcb2.reshape(1, 1))
    return res[:, :1]


# speedup vs baseline: 1.3039x; 1.0057x over previous
import functools

import jax
import jax.numpy as jnp
from jax import lax
from jax.experimental import pallas as pl
from jax.experimental.pallas import tpu as pltpu
from jax.experimental.pallas import tpu_sc as plsc

N = 10000
E = 640000
B = 512
H = 128
PROT = 128

NSC = 2
NSUB = 16
NW = NSC * NSUB
K = 128
NCHUNK = E // K
QR = 80
NQ = N // QR
SEGS_PER_W = B // NW
PCH = 384

_mesh = plsc.VectorSubcoreMesh(core_axis_name="c", subcore_axis_name="s")

_Z16 = functools.partial(jnp.zeros, (16,), jnp.float32)


def _zero_rows(ref, nrows, ncol16):
    def body(r, _):
        for h in range(ncol16):
            ref[r, pl.ds(16 * h, 16)] = _Z16()
        return 0
    lax.fori_loop(0, nrows, body, 0)


@functools.partial(
    pl.kernel,
    out_type=(
        jax.ShapeDtypeStruct((NSC, N, 16), jnp.float32),
        jax.ShapeDtypeStruct((NSC, B, 16), jnp.float32),
    ),
    mesh=_mesh,
    scratch_types=(
        pltpu.VMEM((K,), jnp.int32),
        pltpu.VMEM((K,), jnp.int32),
        pltpu.VMEM((16,), jnp.int32),
        pltpu.VMEM((K, 16), jnp.float32),
        pltpu.VMEM((QR, 16), jnp.float32),
        pltpu.VMEM_SHARED((N, 16), jnp.float32),
        pltpu.VMEM_SHARED((B, 16), jnp.float32),
        pltpu.SemaphoreType.DMA,
        pltpu.SemaphoreType.DMA,
    ),
)
def _sc_counts(dst_hbm, batch_hbm, degw_hbm, cntw_hbm, eb0, eb1, idx16,
               ones, zbuf, Dw, Cw, se0, se1):
    c = lax.axis_index("c")
    s = lax.axis_index("s")
    w = s * NSC + c

    one = jnp.ones((16,), jnp.float32)

    def fill_ones(r, _):
        ones[r, :] = one
        return 0
    lax.fori_loop(0, K, fill_ones, 0)

    _zero_rows(zbuf, QR, 1)
    nq = NQ // NW + jnp.where(w < NQ % NW, 1, 0)

    def zbody(q, _):
        pltpu.sync_copy(zbuf, Dw.at[pl.ds((w + q * NW) * QR, QR)])
        return 0
    lax.fori_loop(0, nq, zbody, 0)
    bper = B // NSUB
    pltpu.sync_copy(zbuf.at[pl.ds(0, bper)], Cw.at[pl.ds(s * bper, bper)])
    plsc.subcore_barrier()

    nch = NCHUNK // NW + jnp.where(w < NCHUNK % NW, 1, 0)

    def eload(j, eb, sem):
        ch = w + j * NW
        return pltpu.async_copy(dst_hbm.at[pl.ds(ch * K, K)], eb, sem)

    def ewait(eb, sem):
        pltpu.make_async_copy(dst_hbm.at[pl.ds(0, K)], eb, sem).wait()

    eload(0, eb0, se0)
    eload(1, eb1, se1)

    def ebody(m, _):
        for par, (ebA, seA) in enumerate(((eb0, se0), (eb1, se1))):
            j = 2 * m + par

            @pl.when(j < nch)
            def _():
                ewait(ebA, seA)
                pltpu.sync_copy(ones, Dw.at[ebA], add=True)

                @pl.when(j + 2 < nch)
                def _():
                    eload(j + 2, ebA, seA)
        return 0
    lax.fori_loop(0, (NCHUNK // NW + 2) // 2, ebody, 0)

    nbfull = N // K
    nb = nbfull // NW + jnp.where(w < nbfull % NW, 1, 0)

    def bbody(j, _):
        ch = w + j * NW
        pltpu.sync_copy(batch_hbm.at[pl.ds(ch * K, K)], eb0)
        pltpu.sync_copy(ones, Cw.at[eb0], add=True)
        return 0
    lax.fori_loop(0, nb, bbody, 0)

    @pl.when(w == nbfull % NW)
    def _():
        pltpu.sync_copy(batch_hbm.at[pl.ds(nbfull * K, N - nbfull * K)], idx16)
        pltpu.sync_copy(ones.at[pl.ds(0, N - nbfull * K)], Cw.at[idx16], add=True)

    plsc.subcore_barrier()

    def obody(q, _):
        r0 = (w + q * NW) * QR
        pltpu.sync_copy(Dw.at[pl.ds(r0, QR)], zbuf)
        pltpu.sync_copy(zbuf, degw_hbm.at[c, pl.ds(r0, QR)])
        return 0
    lax.fori_loop(0, nq, obody, 0)
    pltpu.sync_copy(Cw.at[pl.ds(s * bper, bper)], zbuf.at[pl.ds(0, bper)])
    pltpu.sync_copy(zbuf.at[pl.ds(0, bper)], cntw_hbm.at[c, pl.ds(s * bper, bper)])


@functools.partial(
    pl.kernel,
    out_type=jax.ShapeDtypeStruct((NSC, N, H), jnp.float32),
    mesh=_mesh,
    scratch_types=(
        pltpu.VMEM((K,), jnp.int32),
        pltpu.VMEM((K,), jnp.int32),
        pltpu.VMEM((K,), jnp.int32),
        pltpu.VMEM((K,), jnp.int32),
        pltpu.VMEM((K, H), jnp.float32),
        pltpu.VMEM((K, H), jnp.float32),
        pltpu.VMEM((QR, H), jnp.float32),
        pltpu.VMEM_SHARED((N, H), jnp.float32),
        pltpu.SemaphoreType.DMA,
        pltpu.SemaphoreType.DMA,
        pltpu.SemaphoreType.DMA,
        pltpu.SemaphoreType.DMA,
        pltpu.SemaphoreType.DMA,
        pltpu.SemaphoreType.DMA,
    ),
)
def _sc_prop(g_hbm, src_hbm, dst_hbm, out_hbm, si0, si1, di0, di1,
             rows0, rows1, zbuf, S, ss0, ss1, sd0, sd1, sg0, sg1):
    c = lax.axis_index("c")
    s = lax.axis_index("s")
    w = s * NSC + c

    _zero_rows(zbuf, QR, H // 16)
    nq = NQ // NW + jnp.where(w < NQ % NW, 1, 0)

    def zbody(q, _):
        pltpu.sync_copy(zbuf, S.at[pl.ds((w + q * NW) * QR, QR)])
        return 0
    lax.fori_loop(0, nq, zbody, 0)
    plsc.subcore_barrier()

    nch = NCHUNK // NW + jnp.where(w < NCHUNK % NW, 1, 0)

    def iload(j, si, di, ss, sd):
        ch = w + j * NW
        pltpu.async_copy(src_hbm.at[pl.ds(ch * K, K)], si, ss)
        pltpu.async_copy(dst_hbm.at[pl.ds(ch * K, K)], di, sd)

    def iwait(si, di, ss, sd):
        pltpu.make_async_copy(src_hbm.at[pl.ds(0, K)], si, ss).wait()
        pltpu.make_async_copy(dst_hbm.at[pl.ds(0, K)], di, sd).wait()

    def gwait(rows, sg):
        pltpu.make_async_copy(g_hbm.at[pl.ds(0, K)], rows, sg).wait()

    bufs = ((si0, di0, rows0, ss0, sd0, sg0), (si1, di1, rows1, ss1, sd1, sg1))

    iload(0, si0, di0, ss0, sd0)
    iwait(si0, di0, ss0, sd0)
    pltpu.async_copy(g_hbm.at[si0], rows0, sg0)
    iload(1, si1, di1, ss1, sd1)

    def ebody(m, _):
        for par in range(2):
            j = 2 * m + par
            siA, diA, rowsA, ssA, sdA, sgA = bufs[par]
            siB, diB, rowsB, ssB, sdB, sgB = bufs[1 - par]

            @pl.when(j < nch)
            def _():
                gwait(rowsA, sgA)

                @pl.when(j + 1 < nch)
                def _():
                    iwait(siB, diB, ssB, sdB)
                    pltpu.async_copy(g_hbm.at[siB], rowsB, sgB)

                pltpu.sync_copy(rowsA, S.at[diA], add=True)

                @pl.when(j + 2 < nch)
                def _():
                    iload(j + 2, siA, diA, ssA, sdA)
        return 0
    lax.fori_loop(0, (NCHUNK // NW + 2) // 2, ebody, 0)

    plsc.subcore_barrier()

    def obody(q, _):
        r0 = (w + q * NW) * QR
        pltpu.sync_copy(S.at[pl.ds(r0, QR)], zbuf)
        pltpu.sync_copy(zbuf, out_hbm.at[c, pl.ds(r0, QR)])
        return 0
    lax.fori_loop(0, nq, obody, 0)


@functools.partial(
    pl.kernel,
    out_type=jax.ShapeDtypeStruct((B, H), jnp.float32),
    mesh=_mesh,
    scratch_types=(
        pltpu.VMEM((40,), jnp.int32),
        pltpu.VMEM((PCH, H), jnp.float32),
        pltpu.VMEM((PCH, H), jnp.float32),
        pltpu.VMEM((SEGS_PER_W, H), jnp.float32),
        pltpu.SemaphoreType.DMA,
        pltpu.SemaphoreType.DMA,
    ),
)
def _sc_pool(h_hbm, starts_hbm, out_hbm, stv, buf0, buf1, outbuf, sb0, sb1):
    c = lax.axis_index("c")
    s = lax.axis_index("s")
    w = s * NSC + c

    pltpu.sync_copy(starts_hbm.at[pl.ds(w * SEGS_PER_W, 24)],
                    stv.at[pl.ds(0, 24)])

    _zero_rows(outbuf, SEGS_PER_W, H // 16)

    a0 = stv[pl.ds(0, 16)][0]
    end = stv[pl.ds(16, 16)][0]
    base0 = pl.multiple_of((a0 // 8) * 8, 8)
    nchk = (end - base0 + PCH - 1) // PCH
    bufs = ((buf0, sb0), (buf1, sb1))

    def cstart_of(k2):
        return pl.multiple_of(jnp.minimum(base0 + k2 * PCH, N - PCH), 8)

    def chunk_issue(k2, p):
        pltpu.async_copy(h_hbm.at[pl.ds(cstart_of(k2), PCH)], bufs[p][0],
                         bufs[p][1])

    def chunk_wait(p):
        pltpu.make_async_copy(h_hbm.at[pl.ds(0, PCH)], bufs[p][0],
                              bufs[p][1]).wait()

    @pl.when(nchk > 0)
    def _():
        chunk_issue(0, 0)

    @pl.when(nchk > 1)
    def _():
        chunk_issue(1, 1)

    def ch_body(m, _):
        for p in range(2):
            k2 = 2 * m + p

            @pl.when(k2 < nchk)
            def _():
                chunk_wait(p)
                buf = bufs[p][0]
                cstart = cstart_of(k2)

                def seg_body(j, _2):
                    sv = stv[pl.ds(j, 16)]
                    lo = jnp.maximum(sv[0], cstart)
                    hi = jnp.minimum(sv[1], cstart + PCH)
                    nrows = jnp.maximum(hi - lo, 0)
                    r0 = lo - cstart
                    acc0 = tuple(outbuf[j, pl.ds(16 * h, 16)]
                                 for h in range(H // 16))

                    def row4_body(i, acc):
                        r = r0 + 4 * i
                        for rr in range(4):
                            acc = tuple(
                                jnp.maximum(acc[h],
                                            buf[r + rr, pl.ds(16 * h, 16)])
                                for h in range(H // 16)
                            )
                        return acc

                    acc = lax.fori_loop(0, nrows // 4, row4_body, acc0)

                    def row1_body(i, acc):
                        r = r0 + (nrows // 4) * 4 + i
                        return tuple(
                            jnp.maximum(acc[h], buf[r, pl.ds(16 * h, 16)])
                            for h in range(H // 16)
                        )
                    acc = lax.fori_loop(0, nrows % 4, row1_body, acc)
                    for h in range(H // 16):
                        outbuf[j, pl.ds(16 * h, 16)] = acc[h]
                    return 0
                lax.fori_loop(0, SEGS_PER_W, seg_body, 0)

                @pl.when(k2 + 2 < nchk)
                def _():
                    chunk_issue(k2 + 2, p)
        return 0
    lax.fori_loop(0, (nchk + 1) // 2, ch_body, 0)
    pltpu.sync_copy(outbuf, out_hbm.at[pl.ds(w * SEGS_PER_W, SEGS_PER_W)])


_BLK = 1000


def _prep_mm_body(x_ref, w_ref, dw_ref, cw_ref, o_ref, dinv_ref, starts_ref):
    deg = dw_ref[0, :, 0:1] + dw_ref[1, :, 0:1] + 1.0
    dinv = lax.rsqrt(deg)
    dinv_ref[...] = dinv
    o_ref[...] = jnp.dot(x_ref[...], w_ref[...],
                         preferred_element_type=jnp.float32) * dinv

    @pl.when(pl.program_id(0) == 0)
    def _():
        cnt = cw_ref[0, :, 0:1] + cw_ref[1, :, 0:1]
        row = lax.broadcasted_iota(jnp.int32, (B, B), 0)
        col = lax.broadcasted_iota(jnp.int32, (B, B), 1)
        tril = jnp.where(col < row, 1.0, 0.0)
        st = jnp.dot(tril, cnt, preferred_element_type=jnp.float32)
        starts_ref[pl.ds(0, B)] = st[:, 0].astype(jnp.int32)
        starts_ref[pl.ds(B, 8)] = jnp.full((8,), N, jnp.int32)


def _prep_mm(x, W, degw, cntw):
    return pl.pallas_call(
        _prep_mm_body,
        grid=(N // _BLK,),
        in_specs=[
            pl.BlockSpec((_BLK, H), lambda i: (i, 0)),
            pl.BlockSpec((H, H), lambda i: (0, 0)),
            pl.BlockSpec((NSC, _BLK, 16), lambda i: (0, i, 0)),
            pl.BlockSpec((NSC, B, 16), lambda i: (0, 0, 0)),
        ],
        out_specs=(
            pl.BlockSpec((_BLK, H), lambda i: (i, 0)),
            pl.BlockSpec((_BLK, 1), lambda i: (i, 0)),
            pl.BlockSpec((B + 8,), lambda i: (0,)),
        ),
        out_shape=(
            jax.ShapeDtypeStruct((N, H), jnp.float32),
            jax.ShapeDtypeStruct((N, 1), jnp.float32),
            jax.ShapeDtypeStruct((B + 8,), jnp.int32),
        ),
    )(x, W, degw, cntw)


def _layer_body(S_ref, g_ref, dinv_ref, b_ref, w_ref, o_ref):
    h = jnp.maximum(
        (S_ref[0] + S_ref[1] + g_ref[...]) * dinv_ref[...] + b_ref[...], 0.0)
    o_ref[...] = jnp.dot(h, w_ref[...],
                         preferred_element_type=jnp.float32) * dinv_ref[...]


def _layer(S, g, dinv, b, Wn):
    return pl.pallas_call(
        _layer_body,
        grid=(N // _BLK,),
        in_specs=[
            pl.BlockSpec((NSC, _BLK, H), lambda i: (0, i, 0)),
            pl.BlockSpec((_BLK, H), lambda i: (i, 0)),
            pl.BlockSpec((_BLK, 1), lambda i: (i, 0)),
            pl.BlockSpec((1, H), lambda i: (0, 0)),
            pl.BlockSpec((H, H), lambda i: (0, 0)),
        ],
        out_specs=pl.BlockSpec((_BLK, H), lambda i: (i, 0)),
        out_shape=jax.ShapeDtypeStruct((N, H), jnp.float32),
    )(S, g, dinv, b, Wn)


def _finalh_body(S_ref, g_ref, dinv_ref, b_ref, o_ref):
    o_ref[...] = jnp.maximum(
        (S_ref[0] + S_ref[1] + g_ref[...]) * dinv_ref[...] + b_ref[...], 0.0)


def _finalh(S, g, dinv, b):
    return pl.pallas_call(
        _finalh_body,
        grid=(N // _BLK,),
        in_specs=[
            pl.BlockSpec((NSC, _BLK, H), lambda i: (0, i, 0)),
            pl.BlockSpec((_BLK, H), lambda i: (i, 0)),
            pl.BlockSpec((_BLK, 1), lambda i: (i, 0)),
            pl.BlockSpec((1, H), lambda i: (0, 0)),
        ],
        out_specs=pl.BlockSpec((_BLK, H), lambda i: (i, 0)),
        out_shape=jax.ShapeDtypeStruct((N, H), jnp.float32),
    )(S, g, dinv, b)


def _head_body(p_ref, pr_ref, w1_ref, b1_ref, w2_ref, b2_ref, o_ref):
    z = jnp.dot(p_ref[...], w1_ref[0:H, :], preferred_element_type=jnp.float32)
    z = z + jnp.dot(pr_ref[...], w1_ref[H:, :],
                    preferred_element_type=jnp.float32)
    z = jnp.maximum(z + b1_ref[...], 0.0)
    o_ref[...] = jnp.dot(z, w2_ref[...],
                         preferred_element_type=jnp.float32) + b2_ref[...]


def _head(pooled, prot, fcW1, fcb1, fcW2p, fcb2):
    return pl.pallas_call(
        _head_body,
        out_shape=jax.ShapeDtypeStruct((B, H), jnp.float32),
    )(pooled, prot, fcW1, fcb1, fcW2p, fcb2)


@jax.jit
def kernel(x, edge_index, batch, prot_vec, W0, b0, W1, b1, W2, b2,
           fcW1, fcb1, fcW2, fcb2):
    src = edge_index[0]
    dst = edge_index[1]
    degw, cntw = _sc_counts(dst, batch)
    g0, dinv, starts = _prep_mm(x, W0, degw, cntw)
    S0 = _sc_prop(g0, src, dst)
    g1 = _layer(S0, g0, dinv, b0.reshape(1, H), W1)
    S1 = _sc_prop(g1, src, dst)
    g2 = _layer(S1, g1, dinv, b1.reshape(1, H), W2)
    S2 = _sc_prop(g2, src, dst)
    h3 = _finalh(S2, g2, dinv, b2.reshape(1, H))
    pooled = _sc_pool(h3, starts)
    fcW2p = jnp.pad(fcW2, ((0, 0), (0, H - 1)))
    res = _head(pooled, prot_vec, fcW1, fcb1.reshape(1, 256),
                fcW2p, fcb2.reshape(1, 1))
    return res[:, :1]


# speedup vs baseline: 1.3050x; 1.0008x over previous
import functools

import jax
import jax.numpy as jnp
from jax import lax
from jax.experimental import pallas as pl
from jax.experimental.pallas import tpu as pltpu
from jax.experimental.pallas import tpu_sc as plsc

N = 10000
E = 640000
B = 512
H = 128
PROT = 128

NSC = 2
NSUB = 16
NW = NSC * NSUB
K = 128
NCHUNK = E // K
QR = 80
NQ = N // QR
SEGS_PER_W = B // NW
PCH = 384

_mesh = plsc.VectorSubcoreMesh(core_axis_name="c", subcore_axis_name="s")

_Z16 = functools.partial(jnp.zeros, (16,), jnp.float32)


def _zero_rows(ref, nrows, ncol16):
    def body(r, _):
        for h in range(ncol16):
            ref[r, pl.ds(16 * h, 16)] = _Z16()
        return 0
    lax.fori_loop(0, nrows, body, 0)


@functools.partial(
    pl.kernel,
    out_type=(
        jax.ShapeDtypeStruct((NSC, N, 16), jnp.float32),
        jax.ShapeDtypeStruct((NSC, B, 16), jnp.float32),
    ),
    mesh=_mesh,
    scratch_types=(
        pltpu.VMEM((K,), jnp.int32),
        pltpu.VMEM((K,), jnp.int32),
        pltpu.VMEM((16,), jnp.int32),
        pltpu.VMEM((K, 16), jnp.float32),
        pltpu.VMEM((QR, 16), jnp.float32),
        pltpu.VMEM_SHARED((N, 16), jnp.float32),
        pltpu.VMEM_SHARED((B, 16), jnp.float32),
        pltpu.SemaphoreType.DMA,
        pltpu.SemaphoreType.DMA,
    ),
)
def _sc_counts(dst_hbm, batch_hbm, degw_hbm, cntw_hbm, eb0, eb1, idx16,
               ones, zbuf, Dw, Cw, se0, se1):
    c = lax.axis_index("c")
    s = lax.axis_index("s")
    w = s * NSC + c

    one = jnp.ones((16,), jnp.float32)

    def fill_ones(r, _):
        ones[r, :] = one
        return 0
    lax.fori_loop(0, K, fill_ones, 0)

    _zero_rows(zbuf, QR, 1)
    nq = NQ // NW + jnp.where(w < NQ % NW, 1, 0)

    def zbody(q, _):
        pltpu.sync_copy(zbuf, Dw.at[pl.ds((w + q * NW) * QR, QR)])
        return 0
    lax.fori_loop(0, nq, zbody, 0)
    bper = B // NSUB
    pltpu.sync_copy(zbuf.at[pl.ds(0, bper)], Cw.at[pl.ds(s * bper, bper)])
    plsc.subcore_barrier()

    nch = NCHUNK // NW + jnp.where(w < NCHUNK % NW, 1, 0)

    def eload(j, eb, sem):
        ch = w + j * NW
        return pltpu.async_copy(dst_hbm.at[pl.ds(ch * K, K)], eb, sem)

    def ewait(eb, sem):
        pltpu.make_async_copy(dst_hbm.at[pl.ds(0, K)], eb, sem).wait()

    eload(0, eb0, se0)
    eload(1, eb1, se1)

    def ebody(m, _):
        for par, (ebA, seA) in enumerate(((eb0, se0), (eb1, se1))):
            j = 2 * m + par

            @pl.when(j < nch)
            def _():
                ewait(ebA, seA)
                pltpu.sync_copy(ones, Dw.at[ebA], add=True)

                @pl.when(j + 2 < nch)
                def _():
                    eload(j + 2, ebA, seA)
        return 0
    lax.fori_loop(0, (NCHUNK // NW + 2) // 2, ebody, 0)

    nbfull = N // K
    nb = nbfull // NW + jnp.where(w < nbfull % NW, 1, 0)

    def bbody(j, _):
        ch = w + j * NW
        pltpu.sync_copy(batch_hbm.at[pl.ds(ch * K, K)], eb0)
        pltpu.sync_copy(ones, Cw.at[eb0], add=True)
        return 0
    lax.fori_loop(0, nb, bbody, 0)

    @pl.when(w == nbfull % NW)
    def _():
        pltpu.sync_copy(batch_hbm.at[pl.ds(nbfull * K, N - nbfull * K)], idx16)
        pltpu.sync_copy(ones.at[pl.ds(0, N - nbfull * K)], Cw.at[idx16], add=True)

    plsc.subcore_barrier()

    def obody(q, _):
        r0 = (w + q * NW) * QR
        pltpu.sync_copy(Dw.at[pl.ds(r0, QR)], zbuf)
        pltpu.sync_copy(zbuf, degw_hbm.at[c, pl.ds(r0, QR)])
        return 0
    lax.fori_loop(0, nq, obody, 0)
    pltpu.sync_copy(Cw.at[pl.ds(s * bper, bper)], zbuf.at[pl.ds(0, bper)])
    pltpu.sync_copy(zbuf.at[pl.ds(0, bper)], cntw_hbm.at[c, pl.ds(s * bper, bper)])


@functools.partial(
    pl.kernel,
    out_type=jax.ShapeDtypeStruct((NSC, N, H), jnp.float32),
    mesh=_mesh,
    scratch_types=(
        pltpu.VMEM((K,), jnp.int32),
        pltpu.VMEM((K,), jnp.int32),
        pltpu.VMEM((K,), jnp.int32),
        pltpu.VMEM((K,), jnp.int32),
        pltpu.VMEM((K, H), jnp.float32),
        pltpu.VMEM((K, H), jnp.float32),
        pltpu.VMEM((QR, H), jnp.float32),
        pltpu.VMEM_SHARED((N, H), jnp.float32),
        pltpu.SemaphoreType.DMA,
        pltpu.SemaphoreType.DMA,
        pltpu.SemaphoreType.DMA,
        pltpu.SemaphoreType.DMA,
        pltpu.SemaphoreType.DMA,
        pltpu.SemaphoreType.DMA,
    ),
)
def _sc_prop(g_hbm, src_hbm, dst_hbm, out_hbm, si0, si1, di0, di1,
             rows0, rows1, zbuf, S, ss0, ss1, sd0, sd1, sg0, sg1):
    c = lax.axis_index("c")
    s = lax.axis_index("s")
    w = s * NSC + c

    _zero_rows(zbuf, QR, H // 16)
    nq = NQ // NW + jnp.where(w < NQ % NW, 1, 0)

    def zbody(q, _):
        pltpu.sync_copy(zbuf, S.at[pl.ds((w + q * NW) * QR, QR)])
        return 0
    lax.fori_loop(0, nq, zbody, 0)
    plsc.subcore_barrier()

    nch = NCHUNK // NW + jnp.where(w < NCHUNK % NW, 1, 0)

    def iload(j, si, di, ss, sd):
        ch = w + j * NW
        pltpu.async_copy(src_hbm.at[pl.ds(ch * K, K)], si, ss)
        pltpu.async_copy(dst_hbm.at[pl.ds(ch * K, K)], di, sd)

    def iwait(si, di, ss, sd):
        pltpu.make_async_copy(src_hbm.at[pl.ds(0, K)], si, ss).wait()
        pltpu.make_async_copy(dst_hbm.at[pl.ds(0, K)], di, sd).wait()

    def gwait(rows, sg):
        pltpu.make_async_copy(g_hbm.at[pl.ds(0, K)], rows, sg).wait()

    bufs = ((si0, di0, rows0, ss0, sd0, sg0), (si1, di1, rows1, ss1, sd1, sg1))

    iload(0, si0, di0, ss0, sd0)
    iwait(si0, di0, ss0, sd0)
    pltpu.async_copy(g_hbm.at[si0], rows0, sg0)
    iload(1, si1, di1, ss1, sd1)

    def ebody(m, _):
        for par in range(2):
            j = 2 * m + par
            siA, diA, rowsA, ssA, sdA, sgA = bufs[par]
            siB, diB, rowsB, ssB, sdB, sgB = bufs[1 - par]

            @pl.when(j < nch)
            def _():
                gwait(rowsA, sgA)

                @pl.when(j + 1 < nch)
                def _():
                    iwait(siB, diB, ssB, sdB)
                    pltpu.async_copy(g_hbm.at[siB], rowsB, sgB)

                pltpu.sync_copy(rowsA, S.at[diA], add=True)

                @pl.when(j + 2 < nch)
                def _():
                    iload(j + 2, siA, diA, ssA, sdA)
        return 0
    lax.fori_loop(0, (NCHUNK // NW + 2) // 2, ebody, 0)

    plsc.subcore_barrier()

    def obody(q, _):
        r0 = (w + q * NW) * QR
        pltpu.sync_copy(S.at[pl.ds(r0, QR)], zbuf)
        pltpu.sync_copy(zbuf, out_hbm.at[c, pl.ds(r0, QR)])
        return 0
    lax.fori_loop(0, nq, obody, 0)


@functools.partial(
    pl.kernel,
    out_type=jax.ShapeDtypeStruct((B, H), jnp.float32),
    mesh=_mesh,
    scratch_types=(
        pltpu.VMEM((40,), jnp.int32),
        pltpu.VMEM((PCH, H), jnp.float32),
        pltpu.VMEM((PCH, H), jnp.float32),
        pltpu.VMEM((SEGS_PER_W, H), jnp.float32),
        pltpu.SemaphoreType.DMA,
        pltpu.SemaphoreType.DMA,
    ),
)
def _sc_pool(h_hbm, starts_hbm, out_hbm, stv, buf0, buf1, outbuf, sb0, sb1):
    c = lax.axis_index("c")
    s = lax.axis_index("s")
    w = c * NSUB + s

    pltpu.sync_copy(starts_hbm.at[pl.ds(w * SEGS_PER_W, 24)],
                    stv.at[pl.ds(0, 24)])

    _zero_rows(outbuf, SEGS_PER_W, H // 16)

    a0 = stv[pl.ds(0, 16)][0]
    end = stv[pl.ds(16, 16)][0]
    base0 = pl.multiple_of((a0 // 8) * 8, 8)
    nchk = (end - base0 + PCH - 1) // PCH
    bufs = ((buf0, sb0), (buf1, sb1))

    def cstart_of(k2):
        return pl.multiple_of(jnp.minimum(base0 + k2 * PCH, N - PCH), 8)

    def chunk_issue(k2, p):
        pltpu.async_copy(h_hbm.at[pl.ds(cstart_of(k2), PCH)], bufs[p][0],
                         bufs[p][1])

    def chunk_wait(p):
        pltpu.make_async_copy(h_hbm.at[pl.ds(0, PCH)], bufs[p][0],
                              bufs[p][1]).wait()

    @pl.when(nchk > 0)
    def _():
        chunk_issue(0, 0)

    @pl.when(nchk > 1)
    def _():
        chunk_issue(1, 1)

    def ch_body(m, _):
        for p in range(2):
            k2 = 2 * m + p

            @pl.when(k2 < nchk)
            def _():
                chunk_wait(p)
                buf = bufs[p][0]
                cstart = cstart_of(k2)

                def seg_body(j, _2):
                    sv = stv[pl.ds(j, 16)]
                    lo = jnp.maximum(sv[0], cstart)
                    hi = jnp.minimum(sv[1], cstart + PCH)
                    nrows = jnp.maximum(hi - lo, 0)
                    r0 = lo - cstart
                    acc0 = tuple(outbuf[j, pl.ds(16 * h, 16)]
                                 for h in range(H // 16))

                    def row4_body(i, acc):
                        r = r0 + 4 * i
                        for rr in range(4):
                            acc = tuple(
                                jnp.maximum(acc[h],
                                            buf[r + rr, pl.ds(16 * h, 16)])
                                for h in range(H // 16)
                            )
                        return acc

                    acc = lax.fori_loop(0, nrows // 4, row4_body, acc0)

                    def row1_body(i, acc):
                        r = r0 + (nrows // 4) * 4 + i
                        return tuple(
                            jnp.maximum(acc[h], buf[r, pl.ds(16 * h, 16)])
                            for h in range(H // 16)
                        )
                    acc = lax.fori_loop(0, nrows % 4, row1_body, acc)
                    for h in range(H // 16):
                        outbuf[j, pl.ds(16 * h, 16)] = acc[h]
                    return 0
                lax.fori_loop(0, SEGS_PER_W, seg_body, 0)

                @pl.when(k2 + 2 < nchk)
                def _():
                    chunk_issue(k2 + 2, p)
        return 0
    lax.fori_loop(0, (nchk + 1) // 2, ch_body, 0)
    pltpu.sync_copy(outbuf, out_hbm.at[pl.ds(w * SEGS_PER_W, SEGS_PER_W)])


_BLK = 1000


def _prep_mm_body(x_ref, w_ref, dw_ref, cw_ref, o_ref, dinv_ref, starts_ref):
    deg = dw_ref[0, :, 0:1] + dw_ref[1, :, 0:1] + 1.0
    dinv = lax.rsqrt(deg)
    dinv_ref[...] = dinv
    o_ref[...] = jnp.dot(x_ref[...], w_ref[...],
                         preferred_element_type=jnp.float32) * dinv

    @pl.when(pl.program_id(0) == 0)
    def _():
        cnt = cw_ref[0, :, 0:1] + cw_ref[1, :, 0:1]
        row = lax.broadcasted_iota(jnp.int32, (B, B), 0)
        col = lax.broadcasted_iota(jnp.int32, (B, B), 1)
        tril = jnp.where(col < row, 1.0, 0.0)
        st = jnp.dot(tril, cnt, preferred_element_type=jnp.float32)
        starts_ref[pl.ds(0, B)] = st[:, 0].astype(jnp.int32)
        starts_ref[pl.ds(B, 8)] = jnp.full((8,), N, jnp.int32)


def _prep_mm(x, W, degw, cntw):
    return pl.pallas_call(
        _prep_mm_body,
        grid=(N // _BLK,),
        in_specs=[
            pl.BlockSpec((_BLK, H), lambda i: (i, 0)),
            pl.BlockSpec((H, H), lambda i: (0, 0)),
            pl.BlockSpec((NSC, _BLK, 16), lambda i: (0, i, 0)),
            pl.BlockSpec((NSC, B, 16), lambda i: (0, 0, 0)),
        ],
        out_specs=(
            pl.BlockSpec((_BLK, H), lambda i: (i, 0)),
            pl.BlockSpec((_BLK, 1), lambda i: (i, 0)),
            pl.BlockSpec((B + 8,), lambda i: (0,)),
        ),
        out_shape=(
            jax.ShapeDtypeStruct((N, H), jnp.float32),
            jax.ShapeDtypeStruct((N, 1), jnp.float32),
            jax.ShapeDtypeStruct((B + 8,), jnp.int32),
        ),
    )(x, W, degw, cntw)


def _layer_body(S_ref, g_ref, dinv_ref, b_ref, w_ref, o_ref):
    h = jnp.maximum(
        (S_ref[0] + S_ref[1] + g_ref[...]) * dinv_ref[...] + b_ref[...], 0.0)
    o_ref[...] = jnp.dot(h, w_ref[...],
                         preferred_element_type=jnp.float32) * dinv_ref[...]


def _layer(S, g, dinv, b, Wn):
    return pl.pallas_call(
        _layer_body,
        grid=(N // _BLK,),
        in_specs=[
            pl.BlockSpec((NSC, _BLK, H), lambda i: (0, i, 0)),
            pl.BlockSpec((_BLK, H), lambda i: (i, 0)),
            pl.BlockSpec((_BLK, 1), lambda i: (i, 0)),
            pl.BlockSpec((1, H), lambda i: (0, 0)),
            pl.BlockSpec((H, H), lambda i: (0, 0)),
        ],
        out_specs=pl.BlockSpec((_BLK, H), lambda i: (i, 0)),
        out_shape=jax.ShapeDtypeStruct((N, H), jnp.float32),
    )(S, g, dinv, b, Wn)


def _finalh_body(S_ref, g_ref, dinv_ref, b_ref, o_ref):
    o_ref[...] = jnp.maximum(
        (S_ref[0] + S_ref[1] + g_ref[...]) * dinv_ref[...] + b_ref[...], 0.0)


def _finalh(S, g, dinv, b):
    return pl.pallas_call(
        _finalh_body,
        grid=(N // _BLK,),
        in_specs=[
            pl.BlockSpec((NSC, _BLK, H), lambda i: (0, i, 0)),
            pl.BlockSpec((_BLK, H), lambda i: (i, 0)),
            pl.BlockSpec((_BLK, 1), lambda i: (i, 0)),
            pl.BlockSpec((1, H), lambda i: (0, 0)),
        ],
        out_specs=pl.BlockSpec((_BLK, H), lambda i: (i, 0)),
        out_shape=jax.ShapeDtypeStruct((N, H), jnp.float32),
    )(S, g, dinv, b)


def _head_body(p_ref, pr_ref, w1_ref, b1_ref, w2_ref, b2_ref, o_ref):
    z = jnp.dot(p_ref[...], w1_ref[0:H, :], preferred_element_type=jnp.float32)
    z = z + jnp.dot(pr_ref[...], w1_ref[H:, :],
                    preferred_element_type=jnp.float32)
    z = jnp.maximum(z + b1_ref[...], 0.0)
    o_ref[...] = jnp.dot(z, w2_ref[...],
                         preferred_element_type=jnp.float32) + b2_ref[...]


def _head(pooled, prot, fcW1, fcb1, fcW2p, fcb2):
    return pl.pallas_call(
        _head_body,
        out_shape=jax.ShapeDtypeStruct((B, H), jnp.float32),
    )(pooled, prot, fcW1, fcb1, fcW2p, fcb2)


@jax.jit
def kernel(x, edge_index, batch, prot_vec, W0, b0, W1, b1, W2, b2,
           fcW1, fcb1, fcW2, fcb2):
    src = edge_index[0]
    dst = edge_index[1]
    degw, cntw = _sc_counts(dst, batch)
    g0, dinv, starts = _prep_mm(x, W0, degw, cntw)
    S0 = _sc_prop(g0, src, dst)
    g1 = _layer(S0, g0, dinv, b0.reshape(1, H), W1)
    S1 = _sc_prop(g1, src, dst)
    g2 = _layer(S1, g1, dinv, b1.reshape(1, H), W2)
    S2 = _sc_prop(g2, src, dst)
    h3 = _finalh(S2, g2, dinv, b2.reshape(1, H))
    pooled = _sc_pool(h3, starts)
    fcW2p = jnp.pad(fcW2, ((0, 0), (0, H - 1)))
    res = _head(pooled, prot_vec, fcW1, fcb1.reshape(1, 256),
                fcW2p, fcb2.reshape(1, 1))
    return res[:, :1]
